# baseline (speedup 1.0000x reference)
import functools

import jax
import jax.numpy as jnp
from jax.experimental import pallas as pl
from jax.experimental.pallas import tpu as pltpu


def _lag_kernel(x_ref, w_ref, a_ref, c_ref, out_ref, xcache, pool_acc, gate,
                *, inv_hw, NN, T, tile):
    n = pl.program_id(1)
    t = pl.program_id(2)

    @pl.when(n > 0)
    def _():
        prev = jax.lax.rem(n + 1, 2)
        out_ref[0, 0] = (xcache[prev, :, pl.ds(t * tile, tile)]
                         * gate[...]).astype(out_ref.dtype)

    @pl.when(n < NN)
    def _():
        xb = x_ref[0, 0]
        cur = jax.lax.rem(n, 2)
        xcache[cur, :, pl.ds(t * tile, tile)] = xb

        @pl.when(t == 0)
        def _():
            pool_acc[...] = jnp.zeros_like(pool_acc)

        pool_acc[...] += jnp.sum(xb, axis=1, keepdims=True)

        @pl.when(t == T - 1)
        def _():
            pooled = pool_acc[...] * inv_hw
            conv = jnp.dot(w_ref[...], pooled,
                           preferred_element_type=jnp.float32)
            gate[...] = jax.nn.sigmoid(a_ref[...] * conv + c_ref[...])


def kernel(x, conv1_w, conv1_b, bn_gamma, bn_beta, bn_mean, bn_var, eps=1e-5):
    N, C, H, W = x.shape
    HW = H * W
    NN = N // 2
    tile = 2048 if HW % 2048 == 0 else HW
    T = HW // tile
    x4 = x.reshape(2, NN, C, HW)

    s = bn_gamma * jax.lax.rsqrt(bn_var + eps)
    a_vec = s.reshape(C, 1).astype(jnp.float32)
    c_vec = (s * (conv1_b - bn_mean) + bn_beta).reshape(C, 1).astype(jnp.float32)
    w2 = conv1_w.reshape(C, C).astype(jnp.float32)

    def xmap(h, n, t):
        last = n == NN
        return (h, jnp.where(last, NN - 1, n), 0, jnp.where(last, T - 1, t))

    def omap(h, n, t):
        first = n == 0
        return (h, jnp.where(first, 0, n - 1), 0, jnp.where(first, 0, t))

    body = functools.partial(_lag_kernel, inv_hw=1.0 / HW, NN=NN, T=T,
                             tile=tile)
    cost = pl.CostEstimate(
        flops=int(N * (2 * C * C + 2 * C * HW)),
        transcendentals=int(N * C),
        bytes_accessed=int(2 * N * C * HW * 4 + C * C * 4),
    )
    out4 = pl.pallas_call(
        body,
        out_shape=jax.ShapeDtypeStruct((2, NN, C, HW), jnp.float32),
        grid=(2, NN + 1, T),
        in_specs=[
            pl.BlockSpec((1, 1, C, tile), xmap),
            pl.BlockSpec((C, C), lambda h, n, t: (0, 0)),
            pl.BlockSpec((C, 1), lambda h, n, t: (0, 0)),
            pl.BlockSpec((C, 1), lambda h, n, t: (0, 0)),
        ],
        out_specs=pl.BlockSpec((1, 1, C, tile), omap),
        scratch_shapes=[
            pltpu.VMEM((2, C, HW), jnp.float32),
            pltpu.VMEM((C, 1), jnp.float32),
            pltpu.VMEM((C, 1), jnp.float32),
        ],
        compiler_params=pltpu.CompilerParams(
            dimension_semantics=("parallel", "arbitrary", "arbitrary"),
            vmem_limit_bytes=52 << 20,
        ),
        cost_estimate=cost,
    )(x4, w2, a_vec, c_vec)
    return out4.reshape(N, C, H, W)
